# baseline (speedup 1.0000x reference)
import jax
import jax.numpy as jnp
from jax import lax
from jax.experimental import pallas as pl
from jax.experimental.pallas import tpu as pltpu, tpu_sc as plsc

_B, _S = 1024, 3200
_D = 16
_VOCAB = 1000000
_VPAD = 1000064
_NC, _NS = 2, 16
_NW = _NC * _NS

_GW = 1024
_GWP = _GW + 1
_NG = _VOCAB // _GW
_TAIL = _VOCAB - _NG * _GW
_KMAX = 32

_XR, _XC = 1600, 2048
_RPW = _XR // _NW


def _worker_id():
    return lax.axis_index("s") * _NC + lax.axis_index("c")


def _transpose_rows(src, dst, n_rows, iota):
    def tr_block(j, carry):
        base = jnp.full((16,), j * 32, jnp.int32)
        for u in range(32):
            col = base + u
            row = plsc.load_gather(src, [iota, col])
            dst[pl.ds((j * 32 + u) * _D, _D)] = row
        return carry

    lax.fori_loop(0, n_rows // 32, tr_block, 0)


def _transpose_body(tt_hbm, tail_hbm, lin_hbm, in0, in1, out0, out1, tail_v,
                    si0, si1, so0, so1, st):
    wid = _worker_id()
    iota = lax.iota(jnp.int32, 16)
    ins, outs, si, so = (in0, in1), (out0, out1), (si0, si1), (so0, so1)

    def grp(k):
        return wid + k * _NW

    def in_copy(k, slot):
        return pltpu.make_async_copy(
            tt_hbm.at[:, pl.ds(grp(k) * _GW, _GW)],
            ins[slot].at[:, pl.ds(0, _GW)], si[slot])

    def out_copy(k, slot):
        return pltpu.make_async_copy(
            outs[slot], lin_hbm.at[pl.ds(grp(k) * _GW * _D, _GW * _D)],
            so[slot])

    def steady(k, slot, first):
        g_ok = grp(k) < _NG

        @pl.when(g_ok)
        def _():
            in_copy(k, slot).wait()
            if not first:
                out_copy(k, slot).wait()
            _transpose_rows(ins[slot], outs[slot], _GW, iota)
            out_copy(k, slot).start()

        @pl.when(jnp.logical_and(g_ok, grp(k + 2) < _NG))
        def _():
            in_copy(k + 2, slot).start()

    in_copy(0, 0).start()
    in_copy(1, 1).start()
    steady(0, 0, first=True)
    steady(1, 1, first=True)

    def step(i, carry):
        steady(2 + 2 * i, 0, first=False)
        steady(3 + 2 * i, 1, first=False)
        return carry

    lax.fori_loop(0, (_KMAX - 2) // 2, step, 0)
    for slot in (0, 1):
        out_copy(slot, slot).wait()

    @pl.when(wid == 0)
    def _():
        v0 = _NG * _GW
        cp = pltpu.make_async_copy(tail_hbm, tail_v, st)
        cp.start()
        cp.wait()
        cp = pltpu.make_async_copy(
            tail_v, lin_hbm.at[pl.ds(v0 * _D, _TAIL * _D)], st)
        cp.start()
        cp.wait()


def _gather_body(xp_hbm, tab_hbm, out_hbm, idx0, idx1, rows0, rows1,
                 si0, si1, sg0, sg1, so0, so1):
    wid = _worker_id()
    base = wid * _RPW
    idxs, rows = (idx0, idx1), (rows0, rows1)
    si, sg, so = (si0, si1), (sg0, sg1), (so0, so1)

    def idx_copy(r, slot):
        return pltpu.make_async_copy(xp_hbm.at[base + r], idxs[slot], si[slot])

    def out_copy(r, slot):
        return pltpu.make_async_copy(rows[slot], out_hbm.at[base + r],
                                     so[slot])

    def steady(r, slot, first):
        idx_copy(r, slot).wait()
        if not first:
            out_copy(r, slot).wait()
        pltpu.async_copy(tab_hbm.at[idxs[slot]], rows[slot], sg[slot]).wait()
        out_copy(r, slot).start()
        nxt = jnp.minimum(r + 2, _RPW - 1)
        idx_copy(nxt, slot).start()

    idx_copy(0, 0).start()
    idx_copy(1, 1).start()
    steady(0, 0, first=True)
    steady(1, 1, first=True)

    def step(i, carry):
        steady(2 + 2 * i, 0, first=False)
        steady(3 + 2 * i, 1, first=False)
        return carry

    lax.fori_loop(0, (_RPW - 2) // 2, step, 0)
    for slot in (0, 1):
        out_copy(0, slot).wait()
        idx_copy(0, slot).wait()


def kernel(x, table):
    mesh = plsc.VectorSubcoreMesh(core_axis_name="c", subcore_axis_name="s")

    lin = pl.kernel(
        _transpose_body,
        out_type=jax.ShapeDtypeStruct((_VOCAB * _D,), jnp.float32),
        mesh=mesh,
        scratch_types=[
            pltpu.VMEM((16, _GWP), jnp.float32),
            pltpu.VMEM((16, _GWP), jnp.float32),
            pltpu.VMEM((_GW * _D,), jnp.float32),
            pltpu.VMEM((_GW * _D,), jnp.float32),
            pltpu.VMEM((_TAIL * _D,), jnp.float32),
            pltpu.SemaphoreType.DMA,
            pltpu.SemaphoreType.DMA,
            pltpu.SemaphoreType.DMA,
            pltpu.SemaphoreType.DMA,
            pltpu.SemaphoreType.DMA,
        ],
        compiler_params=pltpu.CompilerParams(needs_layout_passes=False),
    )(table.T, table[_NG * _GW:].reshape(_TAIL * _D))

    out3 = pl.kernel(
        _gather_body,
        out_type=jax.ShapeDtypeStruct((_XR, _XC, _D), jnp.float32),
        mesh=mesh,
        scratch_types=[
            pltpu.VMEM((_XC,), jnp.int32),
            pltpu.VMEM((_XC,), jnp.int32),
            pltpu.VMEM((_XC, _D), jnp.float32),
            pltpu.VMEM((_XC, _D), jnp.float32),
            pltpu.SemaphoreType.DMA,
            pltpu.SemaphoreType.DMA,
            pltpu.SemaphoreType.DMA,
            pltpu.SemaphoreType.DMA,
            pltpu.SemaphoreType.DMA,
            pltpu.SemaphoreType.DMA,
        ],
        compiler_params=pltpu.CompilerParams(use_tc_tiling_on_sc=False),
    )(x.reshape(_XR, _XC), lin.reshape(_VOCAB, _D))

    return out3.reshape(_B, _S // 16, 16 * _D)
